# baseline (speedup 1.0000x reference)
import functools

import jax
import jax.numpy as jnp
from jax import lax
from jax.experimental import pallas as pl
from jax.experimental.pallas import tpu as pltpu
from jax.experimental.pallas import tpu_sc as plsc

NUM_CORES = 2
NUM_SUB = 16
LANES = 16
K = 128
SUPER = 8


def _edge_weights(ea_pad, w_edge):
    e_pad, de = ea_pad.shape
    R = 128
    G = de * R
    rows = e_pad // R
    A = jnp.zeros((G, R), jnp.float32).at[
        jnp.arange(G), jnp.arange(G) // de].set(
        jnp.tile(w_edge.reshape(-1), R))
    ea2 = ea_pad.reshape(rows, G)

    def body(a_ref, m_ref, o_ref):
        o_ref[...] = jnp.dot(a_ref[...], m_ref[...],
                             preferred_element_type=jnp.float32)

    return pl.pallas_call(
        body,
        grid=(1,),
        in_specs=[
            pl.BlockSpec((rows, G), lambda i: (0, 0)),
            pl.BlockSpec((G, R), lambda i: (0, 0)),
        ],
        out_specs=pl.BlockSpec((rows, R), lambda i: (0, 0)),
        out_shape=jax.ShapeDtypeStruct((rows, R), jnp.float32),
    )(ea2, A)


def _post(xf, agg, W, b):
    bv, c = xf.shape
    blk = 2000
    while bv % blk:
        blk //= 2

    def body(x_ref, a_ref, w_ref, b_ref, o_ref):
        h = jnp.dot(a_ref[...], w_ref[...], preferred_element_type=jnp.float32)
        o_ref[...] = x_ref[...] + jnp.tanh(h + b_ref[...])

    return pl.pallas_call(
        body,
        grid=(bv // blk,),
        in_specs=[
            pl.BlockSpec((blk, c), lambda i: (i, 0)),
            pl.BlockSpec((blk, c), lambda i: (i, 0)),
            pl.BlockSpec((c, c), lambda i: (0, 0)),
            pl.BlockSpec((1, c), lambda i: (0, 0)),
        ],
        out_specs=pl.BlockSpec((blk, c), lambda i: (i, 0)),
        out_shape=jax.ShapeDtypeStruct((bv, c), jnp.float32),
    )(xf, agg, W, b.reshape(1, c))


def _make_sc_scatter(B, V, VP, C, NCH):
    BPC = B // NUM_CORES
    RPT = VP // NUM_SUB
    zc = 128
    NSC = NCH // SUPER

    mesh = plsc.VectorSubcoreMesh(
        core_axis_name="c", subcore_axis_name="s",
        num_cores=NUM_CORES, num_subcores=NUM_SUB)

    @functools.partial(
        pl.kernel,
        out_type=jax.ShapeDtypeStruct((B, VP, C), jnp.float32),
        mesh=mesh,
        scratch_types=[
            pltpu.VMEM((2, SUPER, K), jnp.int32),
            pltpu.VMEM((2, SUPER, K), jnp.int32),
            pltpu.VMEM((2, SUPER, K), jnp.float32),
            pltpu.VMEM((K, C), jnp.float32),
            pltpu.VMEM((K, C), jnp.float32),
            pltpu.VMEM_SHARED((VP, C), jnp.float32),
            pltpu.SemaphoreType.DMA,
            pltpu.SemaphoreType.DMA,
            pltpu.SemaphoreType.DMA,
            pltpu.SemaphoreType.DMA,
            pltpu.SemaphoreType.DMA,
        ],
    )
    def sc_kernel(xf_hbm, src_hbm, dst_hbm, ew_hbm, out_hbm,
                  src_v, dst_v, ew_v, rows0, rows1, agg_sh,
                  sem_e, sem_g0, sem_g1, sem_s0, sem_s1):
        cid = lax.axis_index("c")
        sid = lax.axis_index("s")
        rows = (rows0, rows1)
        sem_g = (sem_g0, sem_g1)
        sem_s = (sem_s0, sem_s1)

        def issue_edges(si_next, buf):
            base = pl.multiple_of(si_next * SUPER, SUPER)
            pltpu.async_copy(src_hbm.at[sid, pl.ds(base, SUPER)],
                             src_v.at[buf], sem_e)
            pltpu.async_copy(dst_hbm.at[sid, pl.ds(base, SUPER)],
                             dst_v.at[buf], sem_e)
            pltpu.async_copy(ew_hbm.at[sid, pl.ds(base, SUPER)],
                             ew_v.at[buf], sem_e)

        def wait_edges(si, buf):
            base = pl.multiple_of(si * SUPER, SUPER)
            pltpu.make_async_copy(src_hbm.at[sid, pl.ds(base, SUPER)],
                                  src_v.at[buf], sem_e).wait()
            pltpu.make_async_copy(dst_hbm.at[sid, pl.ds(base, SUPER)],
                                  dst_v.at[buf], sem_e).wait()
            pltpu.make_async_copy(ew_hbm.at[sid, pl.ds(base, SUPER)],
                                  ew_v.at[buf], sem_e).wait()

        issue_edges(0, 0)

        for bi in range(BPC):
            batch = cid * BPC + bi
            off = (cid * BPC + bi) * jnp.int32(V)

            def zero_body(e, _):
                for j in range(C // LANES):
                    rows0[e, pl.ds(j * LANES, LANES)] = jnp.zeros(
                        (LANES,), jnp.float32)
                return 0

            lax.fori_loop(0, zc, zero_body, 0)
            for kk in range(RPT // zc):
                pltpu.sync_copy(
                    rows0.at[pl.ds(0, zc)],
                    agg_sh.at[pl.ds(sid * RPT + kk * zc, zc)])
            plsc.subcore_barrier()

            last_batch = bi == BPC - 1

            def super_body(si, _):
                cur = lax.rem(si, 2)
                wait_edges(si, cur)

                for i in range(SUPER):
                    for j in range(K // LANES):
                        sl = pl.ds(j * LANES, LANES)
                        src_v[cur, i, sl] = src_v[cur, i, sl] + off

                nxt = lax.rem(si + 1, NSC)
                if last_batch:
                    @pl.when(si < NSC - 1)
                    def _():
                        issue_edges(nxt, 1 - cur)
                else:
                    issue_edges(nxt, 1 - cur)

                def gather(ci, p):
                    return pltpu.async_copy(
                        xf_hbm.at[src_v.at[cur, ci]], rows[p], sem_g[p])

                def scale(ci, p):
                    def group_body(g, _):
                        ewl = ew_v[cur, ci, pl.ds(g * LANES, LANES)]
                        for l in range(LANES):
                            e = g * LANES + l
                            s = ewl[l]
                            for j in range(C // LANES):
                                sl = pl.ds(j * LANES, LANES)
                                rows[p][e, sl] = rows[p][e, sl] * s
                        return 0

                    lax.fori_loop(0, K // LANES, group_body, 0)

                def scatter(ci, p):
                    return pltpu.async_copy(
                        rows[p], agg_sh.at[dst_v.at[cur, ci]], sem_s[p],
                        add=True)

                g_desc = [gather(0, 0), gather(1, 1)]
                s_desc = [None, None]
                for ci in range(SUPER):
                    p = ci & 1
                    g_desc[p].wait()
                    scale(ci, p)
                    if 1 <= ci < SUPER - 1:
                        s_desc[1 - p].wait()
                        g_desc[1 - p] = gather(ci + 1, 1 - p)
                    s_desc[p] = scatter(ci, p)
                s_desc[0].wait()
                s_desc[1].wait()
                return 0

            lax.fori_loop(0, NSC, super_body, 0)
            plsc.subcore_barrier()

            for kk in range(RPT // zc):
                r0 = sid * RPT + kk * zc
                pltpu.sync_copy(
                    agg_sh.at[pl.ds(r0, zc)],
                    out_hbm.at[batch, pl.ds(r0, zc)])

    return sc_kernel


def kernel(x, edge_index, edge_attr, W_edge, W, b, T):
    B, V, C = x.shape
    E = edge_index.shape[1] // B

    per_tile = -(-E // NUM_SUB)
    NCH = -(-per_tile // (K * 2 * SUPER)) * 2 * SUPER
    e_pad = NUM_SUB * NCH * K

    src = jnp.pad(edge_index[0, :E], (0, e_pad - E))
    dst = jnp.pad(edge_index[1, :E], (0, e_pad - E))
    ea_pad = jnp.pad(edge_attr[:E], ((0, e_pad - E), (0, 0)))

    ew = _edge_weights(ea_pad, W_edge)

    src3 = src.reshape(NUM_SUB, NCH, K)
    dst3 = dst.reshape(NUM_SUB, NCH, K)
    ew3 = ew.reshape(NUM_SUB, NCH, K)

    xf = x.reshape(B * V, C)
    VP = -(-V // (NUM_SUB * 128)) * (NUM_SUB * 128)
    agg = _make_sc_scatter(B, V, VP, C, NCH)(xf, src3, dst3, ew3)
    out = _post(xf, agg[:, :V, :].reshape(B * V, C), W, b)
    return out.reshape(B, V, C)


# speedup vs baseline: 1.0281x; 1.0281x over previous
import functools

import jax
import jax.numpy as jnp
from jax import lax
from jax.experimental import pallas as pl
from jax.experimental.pallas import tpu as pltpu
from jax.experimental.pallas import tpu_sc as plsc

NUM_CORES = 2
NUM_SUB = 16
LANES = 16
K = 128
SUPER = 8


def _edge_weights(ea_pad, w_edge):
    e_pad, de = ea_pad.shape
    R = 128
    G = de * R
    rows = e_pad // R
    A = jnp.zeros((G, R), jnp.float32).at[
        jnp.arange(G), jnp.arange(G) // de].set(
        jnp.tile(w_edge.reshape(-1), R))
    ea2 = ea_pad.reshape(rows, G)

    def body(a_ref, m_ref, o_ref):
        o_ref[...] = jnp.dot(a_ref[...], m_ref[...],
                             preferred_element_type=jnp.float32)

    return pl.pallas_call(
        body,
        grid=(1,),
        in_specs=[
            pl.BlockSpec((rows, G), lambda i: (0, 0)),
            pl.BlockSpec((G, R), lambda i: (0, 0)),
        ],
        out_specs=pl.BlockSpec((rows, R), lambda i: (0, 0)),
        out_shape=jax.ShapeDtypeStruct((rows, R), jnp.float32),
    )(ea2, A)


def _post(x, agg, W, b):
    B, V, C = x.shape
    blk = 2000
    while V % blk:
        blk //= 2

    def body(x_ref, a_ref, w_ref, b_ref, o_ref):
        h = jnp.dot(a_ref[0], w_ref[...], preferred_element_type=jnp.float32)
        o_ref[0] = x_ref[0] + jnp.tanh(h + b_ref[...])

    return pl.pallas_call(
        body,
        grid=(B, V // blk),
        in_specs=[
            pl.BlockSpec((1, blk, C), lambda bb, i: (bb, i, 0)),
            pl.BlockSpec((1, blk, C), lambda bb, i: (bb, i, 0)),
            pl.BlockSpec((C, C), lambda bb, i: (0, 0)),
            pl.BlockSpec((1, C), lambda bb, i: (0, 0)),
        ],
        out_specs=pl.BlockSpec((1, blk, C), lambda bb, i: (bb, i, 0)),
        out_shape=jax.ShapeDtypeStruct((B, V, C), jnp.float32),
    )(x, agg, W, b.reshape(1, C))


def _make_sc_scatter(B, V, VP, C, NCH):
    BPC = B // NUM_CORES
    RPT = VP // NUM_SUB
    zc = 128
    NSC = NCH // SUPER

    mesh = plsc.VectorSubcoreMesh(
        core_axis_name="c", subcore_axis_name="s",
        num_cores=NUM_CORES, num_subcores=NUM_SUB)

    @functools.partial(
        pl.kernel,
        out_type=jax.ShapeDtypeStruct((B, VP, C), jnp.float32),
        mesh=mesh,
        scratch_types=[
            pltpu.VMEM((2, SUPER, K), jnp.int32),
            pltpu.VMEM((2, SUPER, K), jnp.int32),
            pltpu.VMEM((2, SUPER, K), jnp.float32),
            pltpu.VMEM((K, C), jnp.float32),
            pltpu.VMEM((K, C), jnp.float32),
            pltpu.VMEM_SHARED((VP, C), jnp.float32),
            pltpu.SemaphoreType.DMA,
            pltpu.SemaphoreType.DMA,
            pltpu.SemaphoreType.DMA,
            pltpu.SemaphoreType.DMA,
            pltpu.SemaphoreType.DMA,
            pltpu.SemaphoreType.DMA,
            pltpu.SemaphoreType.DMA,
        ],
    )
    def sc_kernel(xf_hbm, src_hbm, dst_hbm, ew_hbm, out_hbm,
                  src_v, dst_v, ew_v, rows0, rows1, agg_sh,
                  sem_e, sem_g0a, sem_g0b, sem_g1a, sem_g1b,
                  sem_s0, sem_s1):
        cid = lax.axis_index("c")
        sid = lax.axis_index("s")
        rows = (rows0, rows1)
        sem_g = ((sem_g0a, sem_g0b), (sem_g1a, sem_g1b))
        sem_s = (sem_s0, sem_s1)

        def issue_edges(si_next, buf):
            base = pl.multiple_of(si_next * SUPER, SUPER)
            pltpu.async_copy(src_hbm.at[sid, pl.ds(base, SUPER)],
                             src_v.at[buf], sem_e)
            pltpu.async_copy(dst_hbm.at[sid, pl.ds(base, SUPER)],
                             dst_v.at[buf], sem_e)
            pltpu.async_copy(ew_hbm.at[sid, pl.ds(base, SUPER)],
                             ew_v.at[buf], sem_e)

        def wait_edges(si, buf):
            base = pl.multiple_of(si * SUPER, SUPER)
            pltpu.make_async_copy(src_hbm.at[sid, pl.ds(base, SUPER)],
                                  src_v.at[buf], sem_e).wait()
            pltpu.make_async_copy(dst_hbm.at[sid, pl.ds(base, SUPER)],
                                  dst_v.at[buf], sem_e).wait()
            pltpu.make_async_copy(ew_hbm.at[sid, pl.ds(base, SUPER)],
                                  ew_v.at[buf], sem_e).wait()

        issue_edges(0, 0)

        for bi in range(BPC):
            batch = cid * BPC + bi
            off = (cid * BPC + bi) * jnp.int32(V)

            def zero_body(e, _):
                for j in range(C // LANES):
                    rows0[e, pl.ds(j * LANES, LANES)] = jnp.zeros(
                        (LANES,), jnp.float32)
                return 0

            lax.fori_loop(0, zc, zero_body, 0)
            for kk in range(RPT // zc):
                pltpu.sync_copy(
                    rows0.at[pl.ds(0, zc)],
                    agg_sh.at[pl.ds(sid * RPT + kk * zc, zc)])
            plsc.subcore_barrier()

            last_batch = bi == BPC - 1

            def super_body(si, _):
                cur = lax.rem(si, 2)
                wait_edges(si, cur)

                for i in range(SUPER):
                    for j in range(K // LANES):
                        sl = pl.ds(j * LANES, LANES)
                        src_v[cur, i, sl] = src_v[cur, i, sl] + off

                nxt = lax.rem(si + 1, NSC)
                if last_batch:
                    @pl.when(si < NSC - 1)
                    def _():
                        issue_edges(nxt, 1 - cur)
                else:
                    issue_edges(nxt, 1 - cur)

                H = K // 2

                def gather(ci, p):
                    return [
                        pltpu.async_copy(
                            xf_hbm.at[src_v.at[cur, ci, pl.ds(h * H, H)]],
                            rows[p].at[pl.ds(h * H, H)], sem_g[p][h])
                        for h in range(2)
                    ]

                def scale(ci, p, second_half_desc):
                    def group_body(g, _):
                        @pl.when(g == H // LANES)
                        def _():
                            second_half_desc.wait()

                        ewl = ew_v[cur, ci, pl.ds(g * LANES, LANES)]
                        for l in range(LANES):
                            e = g * LANES + l
                            s = ewl[l]
                            for j in range(C // LANES):
                                sl = pl.ds(j * LANES, LANES)
                                rows[p][e, sl] = rows[p][e, sl] * s
                        return 0

                    lax.fori_loop(0, K // LANES, group_body, 0)

                def scatter(ci, p):
                    return pltpu.async_copy(
                        rows[p], agg_sh.at[dst_v.at[cur, ci]], sem_s[p],
                        add=True)

                g_desc = [gather(0, 0), gather(1, 1)]
                s_desc = [None, None]
                for ci in range(SUPER):
                    p = ci & 1
                    g_desc[p][0].wait()
                    scale(ci, p, g_desc[p][1])
                    if 1 <= ci < SUPER - 1:
                        s_desc[1 - p].wait()
                        g_desc[1 - p] = gather(ci + 1, 1 - p)
                    s_desc[p] = scatter(ci, p)
                s_desc[0].wait()
                s_desc[1].wait()
                return 0

            lax.fori_loop(0, NSC, super_body, 0)
            plsc.subcore_barrier()

            for kk in range(RPT // zc):
                r0 = sid * RPT + kk * zc
                pltpu.sync_copy(
                    agg_sh.at[pl.ds(r0, zc)],
                    out_hbm.at[batch, pl.ds(r0, zc)])

    return sc_kernel


def kernel(x, edge_index, edge_attr, W_edge, W, b, T):
    B, V, C = x.shape
    E = edge_index.shape[1] // B

    per_tile = -(-E // NUM_SUB)
    NCH = -(-per_tile // (K * 2 * SUPER)) * 2 * SUPER
    e_pad = NUM_SUB * NCH * K

    src = jnp.pad(edge_index[0, :E], (0, e_pad - E))
    dst = jnp.pad(edge_index[1, :E], (0, e_pad - E))
    ea_pad = jnp.pad(edge_attr[:E], ((0, e_pad - E), (0, 0)))

    ew = _edge_weights(ea_pad, W_edge)

    src3 = src.reshape(NUM_SUB, NCH, K)
    dst3 = dst.reshape(NUM_SUB, NCH, K)
    ew3 = ew.reshape(NUM_SUB, NCH, K)

    xf = x.reshape(B * V, C)
    VP = -(-V // (NUM_SUB * 128)) * (NUM_SUB * 128)
    agg = _make_sc_scatter(B, V, VP, C, NCH)(xf, src3, dst3, ew3)
    return _post(x, agg, W, b)
